# baseline (speedup 1.0000x reference)
import math

import jax
import jax.numpy as jnp
from jax import lax
from jax.experimental import pallas as pl
from jax.experimental.pallas import tpu as pltpu
from jax.experimental.pallas import tpu_sc as plsc

B = 1024
C = 100000
M = 0.4
S = 64.0
EPS = 1e-3
COS_EPS = math.cos(EPS)
COL_BLK = 2048

NC = 2
NS = 16
NW = NC * NS
B_PER_W = B // NW


def _sc_gather_body(win_hbm, labels_hbm, out_hbm,
                    lab_vmem, idx_vmem, win_vmem, val_vmem, sem):
    wid = lax.axis_index("s") * NC + lax.axis_index("c")
    base = wid * B_PER_W
    pltpu.sync_copy(labels_hbm.at[pl.ds(base, B_PER_W)], lab_vmem)
    for k in range(B_PER_W // 16):
        land = lab_vmem[pl.ds(k * 16, 16)]
        row = base + k * 16 + lax.iota(jnp.int32, 16)
        flat_hi = row * (C // 128)
        idx_vmem[pl.ds(k * 16, 16)] = lax.div(row * C + land, 128)
    pltpu.async_copy(win_hbm.at[idx_vmem], win_vmem, sem).wait()
    for k in range(B_PER_W // 16):
        land = lab_vmem[pl.ds(k * 16, 16)]
        row = base + k * 16 + lax.iota(jnp.int32, 16)
        lane = lax.rem(row * C + land, 128)
        val_vmem[pl.ds(k * 16, 16)] = plsc.load_gather(
            win_vmem, [lax.iota(jnp.int32, 16) + k * 16, lane])
    pltpu.sync_copy(val_vmem, out_hbm.at[pl.ds(base, B_PER_W)])


def _sc_gather(logits, labels):
    kfn = pl.kernel(
        _sc_gather_body,
        out_type=jax.ShapeDtypeStruct((B,), jnp.float32),
        mesh=plsc.VectorSubcoreMesh(core_axis_name="c", subcore_axis_name="s"),
        compiler_params=pltpu.CompilerParams(needs_layout_passes=False),
        scratch_types=[
            pltpu.VMEM((B_PER_W,), jnp.int32),
            pltpu.VMEM((B_PER_W,), jnp.int32),
            pltpu.VMEM((B_PER_W, 128), jnp.float32),
            pltpu.VMEM((B_PER_W,), jnp.float32),
            pltpu.SemaphoreType.DMA,
        ],
    )
    return kfn(logits.reshape(B * C // 128, 128), labels)


def _stream_kernel(xt_ref, norms_ref, labels_ref, logits_ref, out_ref, spec_s):
    j = pl.program_id(0)

    @pl.when(j == 0)
    def _prologue():
        safe = jnp.clip(norms_ref[...], 1e-3, 100.0)
        mean = jnp.mean(safe)
        var = jnp.sum((safe - mean) ** 2) / (B - 1)
        std = jnp.sqrt(var)
        ms = (safe - mean) / (std + EPS)
        g = -M * ms
        x = xt_ref[...]
        spec = x * jnp.cos(g) - jnp.sqrt(jnp.maximum(1.0 - x * x, 0.0)) * jnp.sin(g)
        spec = jnp.where(x > jnp.cos(jnp.clip(EPS - g, 0.0, math.pi)), COS_EPS, spec)
        spec = jnp.where(x < jnp.cos(jnp.clip(math.pi - EPS - g, 0.0, math.pi)), -COS_EPS, spec)
        spec_s[...] = S * (spec - (M + M * ms))

    x = logits_ref[...]
    lab_local = labels_ref[...] - j * COL_BLK
    col = jax.lax.broadcasted_iota(jnp.int32, x.shape, 1)
    dense = jnp.clip(S * x, -S * COS_EPS, S * COS_EPS)
    out_ref[...] = jnp.where(col == lab_local, spec_s[...], dense)


@jax.jit
def kernel(logits, norms, labels):
    labels32 = labels.astype(jnp.int32)
    xt = _sc_gather(logits, labels32)
    grid = (C + COL_BLK - 1) // COL_BLK
    return pl.pallas_call(
        _stream_kernel,
        grid=(grid,),
        in_specs=[
            pl.BlockSpec((B, 1), lambda j: (0, 0)),
            pl.BlockSpec((B, 1), lambda j: (0, 0)),
            pl.BlockSpec((B, 1), lambda j: (0, 0)),
            pl.BlockSpec((B, COL_BLK), lambda j: (0, j)),
        ],
        out_specs=pl.BlockSpec((B, COL_BLK), lambda j: (0, j)),
        out_shape=jax.ShapeDtypeStruct((B, C), jnp.float32),
        scratch_shapes=[pltpu.VMEM((B, 1), jnp.float32)],
    )(xt.reshape(B, 1), norms, labels32.reshape(B, 1), logits)


# speedup vs baseline: 1.5762x; 1.5762x over previous
import math

import jax
import jax.numpy as jnp
from jax import lax
from jax.experimental import pallas as pl
from jax.experimental.pallas import tpu as pltpu
from jax.experimental.pallas import tpu_sc as plsc

B = 1024
C = 100000
M = 0.4
S = 64.0
EPS = 1e-3
COS_EPS = math.cos(EPS)
COL_BLK = 2048

NC = 2
NS = 16
NW = NC * NS
B_PER_W = B // NW


def _sc_gather_body(logits_hbm, labels_hbm, out_hbm,
                    lab_vmem, win_vmem, val_vmem, sem):
    wid = lax.axis_index("s") * NC + lax.axis_index("c")
    base = wid * B_PER_W
    pltpu.sync_copy(labels_hbm.at[pl.ds(base, B_PER_W)], lab_vmem)
    copies = []
    for k in range(B_PER_W // 16):
        land = lab_vmem[pl.ds(k * 16, 16)]
        for i in range(16):
            onehot = lax.iota(jnp.int32, 16) == i
            lab = jnp.sum(jnp.where(onehot, land, 0))
            start8 = pl.multiple_of((lab // 8) * 8, 8)
            r = k * 16 + i
            cp = pltpu.make_async_copy(
                logits_hbm.at[base + r, pl.ds(start8, 8)],
                win_vmem.at[pl.ds(r * 8, 8)],
                sem,
            )
            cp.start()
            copies.append(cp)
    for cp in copies:
        cp.wait()
    for k in range(B_PER_W // 16):
        land = lab_vmem[pl.ds(k * 16, 16)]
        idx = (lax.iota(jnp.int32, 16) + k * 16) * 8 + (land & 7)
        val_vmem[pl.ds(k * 16, 16)] = plsc.load_gather(win_vmem, [idx])
    pltpu.sync_copy(val_vmem, out_hbm.at[pl.ds(base, B_PER_W)])


def _sc_gather(logits, labels):
    kfn = pl.kernel(
        _sc_gather_body,
        out_type=jax.ShapeDtypeStruct((B,), jnp.float32),
        mesh=plsc.VectorSubcoreMesh(core_axis_name="c", subcore_axis_name="s"),
        compiler_params=pltpu.CompilerParams(needs_layout_passes=False),
        scratch_types=[
            pltpu.VMEM((B_PER_W,), jnp.int32),
            pltpu.VMEM((B_PER_W * 8,), jnp.float32),
            pltpu.VMEM((B_PER_W,), jnp.float32),
            pltpu.SemaphoreType.DMA,
        ],
    )
    return kfn(logits, labels)


def _stream_kernel(xt_ref, norms_ref, labels_ref, logits_ref, out_ref, spec_s):
    j = pl.program_id(0)

    @pl.when(j == 0)
    def _prologue():
        safe = jnp.clip(norms_ref[...], 1e-3, 100.0)
        mean = jnp.mean(safe)
        var = jnp.sum((safe - mean) ** 2) / (B - 1)
        std = jnp.sqrt(var)
        ms = (safe - mean) / (std + EPS)
        g = -M * ms
        x = xt_ref[...]
        spec = x * jnp.cos(g) - jnp.sqrt(jnp.maximum(1.0 - x * x, 0.0)) * jnp.sin(g)
        spec = jnp.where(x > jnp.cos(jnp.clip(EPS - g, 0.0, math.pi)), COS_EPS, spec)
        spec = jnp.where(x < jnp.cos(jnp.clip(math.pi - EPS - g, 0.0, math.pi)), -COS_EPS, spec)
        spec_s[...] = S * (spec - (M + M * ms))

    x = logits_ref[...]
    lab_local = labels_ref[...] - j * COL_BLK
    col = jax.lax.broadcasted_iota(jnp.int32, x.shape, 1)
    dense = jnp.clip(S * x, -S * COS_EPS, S * COS_EPS)
    out_ref[...] = jnp.where(col == lab_local, spec_s[...], dense)


@jax.jit
def kernel(logits, norms, labels):
    labels32 = labels.astype(jnp.int32)
    xt = _sc_gather(logits, labels32)
    grid = (C + COL_BLK - 1) // COL_BLK
    return pl.pallas_call(
        _stream_kernel,
        grid=(grid,),
        in_specs=[
            pl.BlockSpec((B, 1), lambda j: (0, 0)),
            pl.BlockSpec((B, 1), lambda j: (0, 0)),
            pl.BlockSpec((B, 1), lambda j: (0, 0)),
            pl.BlockSpec((B, COL_BLK), lambda j: (0, j)),
        ],
        out_specs=pl.BlockSpec((B, COL_BLK), lambda j: (0, j)),
        out_shape=jax.ShapeDtypeStruct((B, C), jnp.float32),
        scratch_shapes=[pltpu.VMEM((B, 1), jnp.float32)],
    )(xt.reshape(B, 1), norms, labels32.reshape(B, 1), logits)


# speedup vs baseline: 1.5771x; 1.0006x over previous
import math

import jax
import jax.numpy as jnp
from jax import lax
from jax.experimental import pallas as pl
from jax.experimental.pallas import tpu as pltpu
from jax.experimental.pallas import tpu_sc as plsc

B = 1024
C = 100000
M = 0.4
S = 64.0
EPS = 1e-3
COS_EPS = math.cos(EPS)
COL_BLK = 2560

NC = 2
NS = 16
NW = NC * NS
B_PER_W = B // NW


def _sc_gather_body(logits_hbm, labels_hbm, out_hbm,
                    lab_vmem, win_vmem, val_vmem, sem):
    wid = lax.axis_index("s") * NC + lax.axis_index("c")
    base = wid * B_PER_W
    pltpu.sync_copy(labels_hbm.at[pl.ds(base, B_PER_W)], lab_vmem)
    copies = []
    for k in range(B_PER_W // 16):
        land = lab_vmem[pl.ds(k * 16, 16)]
        for i in range(16):
            onehot = lax.iota(jnp.int32, 16) == i
            lab = jnp.sum(jnp.where(onehot, land, 0))
            start8 = pl.multiple_of((lab // 8) * 8, 8)
            r = k * 16 + i
            cp = pltpu.make_async_copy(
                logits_hbm.at[base + r, pl.ds(start8, 8)],
                win_vmem.at[pl.ds(r * 8, 8)],
                sem,
            )
            cp.start()
            copies.append(cp)
    for cp in copies:
        cp.wait()
    for k in range(B_PER_W // 16):
        land = lab_vmem[pl.ds(k * 16, 16)]
        idx = (lax.iota(jnp.int32, 16) + k * 16) * 8 + (land & 7)
        val_vmem[pl.ds(k * 16, 16)] = plsc.load_gather(win_vmem, [idx])
    pltpu.sync_copy(val_vmem, out_hbm.at[pl.ds(base, B_PER_W)])


def _sc_gather(logits, labels):
    kfn = pl.kernel(
        _sc_gather_body,
        out_type=jax.ShapeDtypeStruct((B,), jnp.float32),
        mesh=plsc.VectorSubcoreMesh(core_axis_name="c", subcore_axis_name="s"),
        compiler_params=pltpu.CompilerParams(needs_layout_passes=False),
        scratch_types=[
            pltpu.VMEM((B_PER_W,), jnp.int32),
            pltpu.VMEM((B_PER_W * 8,), jnp.float32),
            pltpu.VMEM((B_PER_W,), jnp.float32),
            pltpu.SemaphoreType.DMA,
        ],
    )
    return kfn(logits, labels)


def _stream_kernel(xt_ref, norms_ref, labels_ref, logits_ref, out_ref, spec_s):
    j = pl.program_id(0)

    @pl.when(j == 0)
    def _prologue():
        safe = jnp.clip(norms_ref[...], 1e-3, 100.0)
        mean = jnp.mean(safe)
        var = jnp.sum((safe - mean) ** 2) / (B - 1)
        std = jnp.sqrt(var)
        ms = (safe - mean) / (std + EPS)
        g = -M * ms
        x = xt_ref[...]
        spec = x * jnp.cos(g) - jnp.sqrt(jnp.maximum(1.0 - x * x, 0.0)) * jnp.sin(g)
        spec = jnp.where(x > jnp.cos(jnp.clip(EPS - g, 0.0, math.pi)), COS_EPS, spec)
        spec = jnp.where(x < jnp.cos(jnp.clip(math.pi - EPS - g, 0.0, math.pi)), -COS_EPS, spec)
        spec_s[...] = S * (spec - (M + M * ms))

    x = logits_ref[...]
    lab_local = labels_ref[...] - j * COL_BLK
    col = jax.lax.broadcasted_iota(jnp.int32, x.shape, 1)
    dense = jnp.clip(S * x, -S * COS_EPS, S * COS_EPS)
    out_ref[...] = jnp.where(col == lab_local, spec_s[...], dense)


@jax.jit
def kernel(logits, norms, labels):
    labels32 = labels.astype(jnp.int32)
    xt = _sc_gather(logits, labels32)
    grid = (C + COL_BLK - 1) // COL_BLK
    return pl.pallas_call(
        _stream_kernel,
        grid=(grid,),
        in_specs=[
            pl.BlockSpec((B, 1), lambda j: (0, 0)),
            pl.BlockSpec((B, 1), lambda j: (0, 0)),
            pl.BlockSpec((B, 1), lambda j: (0, 0)),
            pl.BlockSpec((B, COL_BLK), lambda j: (0, j)),
        ],
        out_specs=pl.BlockSpec((B, COL_BLK), lambda j: (0, j)),
        out_shape=jax.ShapeDtypeStruct((B, C), jnp.float32),
        scratch_shapes=[pltpu.VMEM((B, 1), jnp.float32)],
    )(xt.reshape(B, 1), norms, labels32.reshape(B, 1), logits)


# speedup vs baseline: 1.5799x; 1.0017x over previous
import math

import jax
import jax.numpy as jnp
from jax import lax
from jax.experimental import pallas as pl
from jax.experimental.pallas import tpu as pltpu
from jax.experimental.pallas import tpu_sc as plsc

B = 1024
C = 100000
M = 0.4
S = 64.0
EPS = 1e-3
COS_EPS = math.cos(EPS)
COL_BLK = 3072

NC = 2
NS = 16
NW = NC * NS
B_PER_W = B // NW


def _sc_gather_body(logits_hbm, labels_hbm, out_hbm,
                    lab_vmem, win_vmem, val_vmem, sem):
    wid = lax.axis_index("s") * NC + lax.axis_index("c")
    base = wid * B_PER_W
    pltpu.sync_copy(labels_hbm.at[pl.ds(base, B_PER_W)], lab_vmem)
    copies = []
    for k in range(B_PER_W // 16):
        land = lab_vmem[pl.ds(k * 16, 16)]
        for i in range(16):
            onehot = lax.iota(jnp.int32, 16) == i
            lab = jnp.sum(jnp.where(onehot, land, 0))
            start8 = pl.multiple_of((lab // 8) * 8, 8)
            r = k * 16 + i
            cp = pltpu.make_async_copy(
                logits_hbm.at[base + r, pl.ds(start8, 8)],
                win_vmem.at[pl.ds(r * 8, 8)],
                sem,
            )
            cp.start()
            copies.append(cp)
    for cp in copies:
        cp.wait()
    for k in range(B_PER_W // 16):
        land = lab_vmem[pl.ds(k * 16, 16)]
        idx = (lax.iota(jnp.int32, 16) + k * 16) * 8 + (land & 7)
        val_vmem[pl.ds(k * 16, 16)] = plsc.load_gather(win_vmem, [idx])
    pltpu.sync_copy(val_vmem, out_hbm.at[pl.ds(base, B_PER_W)])


def _sc_gather(logits, labels):
    kfn = pl.kernel(
        _sc_gather_body,
        out_type=jax.ShapeDtypeStruct((B,), jnp.float32),
        mesh=plsc.VectorSubcoreMesh(core_axis_name="c", subcore_axis_name="s"),
        compiler_params=pltpu.CompilerParams(needs_layout_passes=False),
        scratch_types=[
            pltpu.VMEM((B_PER_W,), jnp.int32),
            pltpu.VMEM((B_PER_W * 8,), jnp.float32),
            pltpu.VMEM((B_PER_W,), jnp.float32),
            pltpu.SemaphoreType.DMA,
        ],
    )
    return kfn(logits, labels)


def _stream_kernel(xt_ref, norms_ref, labels_ref, logits_ref, out_ref, spec_s):
    j = pl.program_id(0)

    @pl.when(j == 0)
    def _prologue():
        safe = jnp.clip(norms_ref[...], 1e-3, 100.0)
        mean = jnp.mean(safe)
        var = jnp.sum((safe - mean) ** 2) / (B - 1)
        std = jnp.sqrt(var)
        ms = (safe - mean) / (std + EPS)
        g = -M * ms
        x = xt_ref[...]
        spec = x * jnp.cos(g) - jnp.sqrt(jnp.maximum(1.0 - x * x, 0.0)) * jnp.sin(g)
        spec = jnp.where(x > jnp.cos(jnp.clip(EPS - g, 0.0, math.pi)), COS_EPS, spec)
        spec = jnp.where(x < jnp.cos(jnp.clip(math.pi - EPS - g, 0.0, math.pi)), -COS_EPS, spec)
        spec_s[...] = S * (spec - (M + M * ms))

    x = logits_ref[...]
    lab_local = labels_ref[...] - j * COL_BLK
    col = jax.lax.broadcasted_iota(jnp.int32, x.shape, 1)
    dense = jnp.clip(S * x, -S * COS_EPS, S * COS_EPS)
    out_ref[...] = jnp.where(col == lab_local, spec_s[...], dense)


@jax.jit
def kernel(logits, norms, labels):
    labels32 = labels.astype(jnp.int32)
    xt = _sc_gather(logits, labels32)
    grid = (C + COL_BLK - 1) // COL_BLK
    return pl.pallas_call(
        _stream_kernel,
        grid=(grid,),
        in_specs=[
            pl.BlockSpec((B, 1), lambda j: (0, 0)),
            pl.BlockSpec((B, 1), lambda j: (0, 0)),
            pl.BlockSpec((B, 1), lambda j: (0, 0)),
            pl.BlockSpec((B, COL_BLK), lambda j: (0, j)),
        ],
        out_specs=pl.BlockSpec((B, COL_BLK), lambda j: (0, j)),
        out_shape=jax.ShapeDtypeStruct((B, C), jnp.float32),
        scratch_shapes=[pltpu.VMEM((B, 1), jnp.float32)],
    )(xt.reshape(B, 1), norms, labels32.reshape(B, 1), logits)
